# baseline (speedup 1.0000x reference)
import functools

import jax
import jax.numpy as jnp
from jax import lax
from jax.experimental import pallas as pl
from jax.experimental.pallas import tpu as pltpu
from jax.experimental.pallas import tpu_sc as plsc

NC = 2
NS = 16
NW = NC * NS
L = 16

BATCH = 16384
DIM = 64
PDIM = 128
NROW = 100000
RPW = BATCH // NW
C = 64
NCHUNK = RPW // C

TBLK = 512


def _transpose_body(in_ref, out_ref):
    out_ref[:, :DIM] = in_ref[...].T


def _pad_rows(table):
    nblk = (NROW + TBLK - 1) // TBLK
    return pl.pallas_call(
        _transpose_body,
        grid=(nblk,),
        in_specs=[pl.BlockSpec((DIM, TBLK), lambda i: (0, i))],
        out_specs=pl.BlockSpec((TBLK, PDIM), lambda i: (i, 0)),
        out_shape=jax.ShapeDtypeStruct((NROW, PDIM), jnp.float32),
    )(table.T)


def _tri_vec_body(hidx_hbm, ridx_hbm, tidx_hbm,
                  e1_hbm, e2_hbm, e3_hbm, r1_hbm, r2_hbm, r3_hbm,
                  out_hbm,
                  hid_v, rid_v, tid_v,
                  h1_v, h2_v, h3_v, t1_v, t2_v, t3_v, rr1_v, rr2_v, rr3_v,
                  out_v, sem):
    wid = lax.axis_index("s") * NC + lax.axis_index("c")
    base = wid * RPW

    pltpu.sync_copy(hidx_hbm.at[pl.ds(base, RPW)], hid_v)
    pltpu.sync_copy(ridx_hbm.at[pl.ds(base, RPW)], rid_v)
    pltpu.sync_copy(tidx_hbm.at[pl.ds(base, RPW)], tid_v)

    lanes = lax.iota(jnp.int32, L)
    lane0 = lanes == 0

    for c in range(NCHUNK):
        off = c * C
        hid = hid_v.at[pl.ds(off, C)]
        rid = rid_v.at[pl.ds(off, C)]
        tid = tid_v.at[pl.ds(off, C)]
        copies = [
            pltpu.async_copy(e1_hbm.at[hid], h1_v, sem),
            pltpu.async_copy(e2_hbm.at[hid], h2_v, sem),
            pltpu.async_copy(e3_hbm.at[hid], h3_v, sem),
            pltpu.async_copy(e1_hbm.at[tid], t1_v, sem),
            pltpu.async_copy(e2_hbm.at[tid], t2_v, sem),
            pltpu.async_copy(e3_hbm.at[tid], t3_v, sem),
            pltpu.async_copy(r1_hbm.at[rid], rr1_v, sem),
            pltpu.async_copy(r2_hbm.at[rid], rr2_v, sem),
            pltpu.async_copy(r3_hbm.at[rid], rr3_v, sem),
        ]
        for cp in copies:
            cp.wait()

        def row_body(i, _, off=off):
            acc = jnp.zeros((L,), jnp.float32)
            for k in range(DIM // L):
                sl = pl.ds(k * L, L)
                acc = acc + h1_v[i, sl] * rr1_v[i, sl] * t3_v[i, sl]
                acc = acc + h2_v[i, sl] * rr2_v[i, sl] * t2_v[i, sl]
                acc = acc + h3_v[i, sl] * rr3_v[i, sl] * t1_v[i, sl]
            s = jnp.sum(acc)
            plsc.store_scatter(out_v,
                               [jnp.full((L,), off, jnp.int32) + i],
                               jnp.full((L,), s, jnp.float32),
                               mask=lane0)
            return 0

        lax.fori_loop(0, C, row_body, 0)

    pltpu.sync_copy(out_v, out_hbm.at[pl.ds(base, RPW)])


@jax.jit
def _tri_vec(h_idx, r_idx, t_idx, ent_1, ent_2, ent_3, rel_1, rel_2, rel_3):
    mesh = plsc.VectorSubcoreMesh(core_axis_name="c", subcore_axis_name="s",
                                  num_cores=NC, num_subcores=NS)
    f = pl.kernel(
        _tri_vec_body,
        out_type=jax.ShapeDtypeStruct((BATCH,), jnp.float32),
        mesh=mesh,
        scratch_types=[
            pltpu.VMEM((RPW,), jnp.int32),
            pltpu.VMEM((RPW,), jnp.int32),
            pltpu.VMEM((RPW,), jnp.int32),
        ] + [pltpu.VMEM((C, PDIM), jnp.float32)] * 9 + [
            pltpu.VMEM((RPW,), jnp.float32),
            pltpu.SemaphoreType.DMA,
        ],
        compiler_params=pltpu.CompilerParams(needs_layout_passes=False),
    )
    return f(h_idx, r_idx, t_idx,
             _pad_rows(ent_1), _pad_rows(ent_2), _pad_rows(ent_3),
             _pad_rows(rel_1), _pad_rows(rel_2), _pad_rows(rel_3))


def kernel(data, ent_1, ent_2, ent_3, rel_1, rel_2, rel_3):
    h_idx = data[:, 0]
    r_idx = data[:, 1]
    t_idx = data[:, 2]
    return _tri_vec(h_idx, r_idx, t_idx, ent_1, ent_2, ent_3,
                    rel_1, rel_2, rel_3)


# speedup vs baseline: 2.1610x; 2.1610x over previous
import functools

import jax
import jax.numpy as jnp
from jax import lax
from jax.experimental import pallas as pl
from jax.experimental.pallas import tpu as pltpu
from jax.experimental.pallas import tpu_sc as plsc

NC = 2
NS = 16
NW = NC * NS
L = 16

BATCH = 16384
DIM = 64
PDIM = 128
NROW = 100000
RPW = BATCH // NW
C = 64
NCHUNK = RPW // C

TBLK = 2048


def _transpose_body(in_ref, out_ref):
    rows = lax.broadcasted_iota(jnp.int32, (DIM, DIM), 0)
    cols = lax.broadcasted_iota(jnp.int32, (DIM, DIM), 1)
    eye = (rows == cols).astype(jnp.float32)
    out_ref[:, :DIM] = lax.dot_general(
        in_ref[...], eye, (((0,), (0,)), ((), ())),
        preferred_element_type=jnp.float32)


def _pad_rows(table):
    nblk = (NROW + TBLK - 1) // TBLK
    return pl.pallas_call(
        _transpose_body,
        grid=(nblk,),
        in_specs=[pl.BlockSpec((DIM, TBLK), lambda i: (0, i))],
        out_specs=pl.BlockSpec((TBLK, PDIM), lambda i: (i, 0)),
        out_shape=jax.ShapeDtypeStruct((NROW, PDIM), jnp.float32),
    )(table.T)


def _tri_vec_body(hidx_hbm, ridx_hbm, tidx_hbm,
                  e1_hbm, e2_hbm, e3_hbm, r1_hbm, r2_hbm, r3_hbm,
                  out_hbm,
                  hid_v, rid_v, tid_v,
                  h1_v, h2_v, h3_v, t1_v, t2_v, t3_v, rr1_v, rr2_v, rr3_v,
                  out_v, sem):
    wid = lax.axis_index("s") * NC + lax.axis_index("c")
    base = wid * RPW

    pltpu.sync_copy(hidx_hbm.at[pl.ds(base, RPW)], hid_v)
    pltpu.sync_copy(ridx_hbm.at[pl.ds(base, RPW)], rid_v)
    pltpu.sync_copy(tidx_hbm.at[pl.ds(base, RPW)], tid_v)

    lanes = lax.iota(jnp.int32, L)
    lane0 = lanes == 0

    for c in range(NCHUNK):
        off = c * C
        hid = hid_v.at[pl.ds(off, C)]
        rid = rid_v.at[pl.ds(off, C)]
        tid = tid_v.at[pl.ds(off, C)]
        copies = [
            pltpu.async_copy(e1_hbm.at[hid], h1_v, sem),
            pltpu.async_copy(e2_hbm.at[hid], h2_v, sem),
            pltpu.async_copy(e3_hbm.at[hid], h3_v, sem),
            pltpu.async_copy(e1_hbm.at[tid], t1_v, sem),
            pltpu.async_copy(e2_hbm.at[tid], t2_v, sem),
            pltpu.async_copy(e3_hbm.at[tid], t3_v, sem),
            pltpu.async_copy(r1_hbm.at[rid], rr1_v, sem),
            pltpu.async_copy(r2_hbm.at[rid], rr2_v, sem),
            pltpu.async_copy(r3_hbm.at[rid], rr3_v, sem),
        ]
        for cp in copies:
            cp.wait()

        def row_body(i, _, off=off):
            acc = jnp.zeros((L,), jnp.float32)
            for k in range(DIM // L):
                sl = pl.ds(k * L, L)
                acc = acc + h1_v[i, sl] * rr1_v[i, sl] * t3_v[i, sl]
                acc = acc + h2_v[i, sl] * rr2_v[i, sl] * t2_v[i, sl]
                acc = acc + h3_v[i, sl] * rr3_v[i, sl] * t1_v[i, sl]
            s = jnp.sum(acc)
            plsc.store_scatter(out_v,
                               [jnp.full((L,), off, jnp.int32) + i],
                               jnp.full((L,), s, jnp.float32),
                               mask=lane0)
            return 0

        lax.fori_loop(0, C, row_body, 0)

    pltpu.sync_copy(out_v, out_hbm.at[pl.ds(base, RPW)])


@jax.jit
def _tri_vec(h_idx, r_idx, t_idx, ent_1, ent_2, ent_3, rel_1, rel_2, rel_3):
    mesh = plsc.VectorSubcoreMesh(core_axis_name="c", subcore_axis_name="s",
                                  num_cores=NC, num_subcores=NS)
    f = pl.kernel(
        _tri_vec_body,
        out_type=jax.ShapeDtypeStruct((BATCH,), jnp.float32),
        mesh=mesh,
        scratch_types=[
            pltpu.VMEM((RPW,), jnp.int32),
            pltpu.VMEM((RPW,), jnp.int32),
            pltpu.VMEM((RPW,), jnp.int32),
        ] + [pltpu.VMEM((C, PDIM), jnp.float32)] * 9 + [
            pltpu.VMEM((RPW,), jnp.float32),
            pltpu.SemaphoreType.DMA,
        ],
        compiler_params=pltpu.CompilerParams(needs_layout_passes=False),
    )
    return f(h_idx, r_idx, t_idx,
             _pad_rows(ent_1), _pad_rows(ent_2), _pad_rows(ent_3),
             _pad_rows(rel_1), _pad_rows(rel_2), _pad_rows(rel_3))


def kernel(data, ent_1, ent_2, ent_3, rel_1, rel_2, rel_3):
    h_idx = data[:, 0]
    r_idx = data[:, 1]
    t_idx = data[:, 2]
    return _tri_vec(h_idx, r_idx, t_idx, ent_1, ent_2, ent_3,
                    rel_1, rel_2, rel_3)


# speedup vs baseline: 2.7348x; 1.2656x over previous
import functools

import jax
import jax.numpy as jnp
from jax import lax
from jax.experimental import pallas as pl
from jax.experimental.pallas import tpu as pltpu
from jax.experimental.pallas import tpu_sc as plsc

NC = 2
NS = 16
NW = NC * NS
L = 16

BATCH = 16384
DIM = 64
PDIM = 128
NROW = 100000
RPW = BATCH // NW
C = 64
NCHUNK = RPW // C

TBLK = 4096


def _transpose_body(in_ref, out_ref):
    rows = lax.broadcasted_iota(jnp.int32, (DIM, PDIM), 0)
    cols = lax.broadcasted_iota(jnp.int32, (DIM, PDIM), 1)
    eye = (rows == cols).astype(jnp.float32)
    out_ref[...] = lax.dot_general(
        in_ref[...], eye, (((0,), (0,)), ((), ())),
        preferred_element_type=jnp.float32)


def _pad_rows(table):
    nblk = (NROW + TBLK - 1) // TBLK
    return pl.pallas_call(
        _transpose_body,
        grid=(nblk,),
        in_specs=[pl.BlockSpec((DIM, TBLK), lambda i: (0, i))],
        out_specs=pl.BlockSpec((TBLK, PDIM), lambda i: (i, 0)),
        out_shape=jax.ShapeDtypeStruct((NROW, PDIM), jnp.float32),
    )(table.T)


def _tri_vec_body(hidx_hbm, ridx_hbm, tidx_hbm,
                  e1_hbm, e2_hbm, e3_hbm, r1_hbm, r2_hbm, r3_hbm,
                  out_hbm,
                  hid_v, rid_v, tid_v,
                  h1_v, h2_v, h3_v, t1_v, t2_v, t3_v, rr1_v, rr2_v, rr3_v,
                  out_v, sem):
    wid = lax.axis_index("s") * NC + lax.axis_index("c")
    base = wid * RPW

    pltpu.sync_copy(hidx_hbm.at[pl.ds(base, RPW)], hid_v)
    pltpu.sync_copy(ridx_hbm.at[pl.ds(base, RPW)], rid_v)
    pltpu.sync_copy(tidx_hbm.at[pl.ds(base, RPW)], tid_v)

    lanes = lax.iota(jnp.int32, L)
    lane0 = lanes == 0

    for c in range(NCHUNK):
        off = c * C
        hid = hid_v.at[pl.ds(off, C)]
        rid = rid_v.at[pl.ds(off, C)]
        tid = tid_v.at[pl.ds(off, C)]
        copies = [
            pltpu.async_copy(e1_hbm.at[hid], h1_v, sem),
            pltpu.async_copy(e2_hbm.at[hid], h2_v, sem),
            pltpu.async_copy(e3_hbm.at[hid], h3_v, sem),
            pltpu.async_copy(e1_hbm.at[tid], t1_v, sem),
            pltpu.async_copy(e2_hbm.at[tid], t2_v, sem),
            pltpu.async_copy(e3_hbm.at[tid], t3_v, sem),
            pltpu.async_copy(r1_hbm.at[rid], rr1_v, sem),
            pltpu.async_copy(r2_hbm.at[rid], rr2_v, sem),
            pltpu.async_copy(r3_hbm.at[rid], rr3_v, sem),
        ]
        for cp in copies:
            cp.wait()

        def row_body(i, _, off=off):
            acc = jnp.zeros((L,), jnp.float32)
            for k in range(DIM // L):
                sl = pl.ds(k * L, L)
                acc = acc + h1_v[i, sl] * rr1_v[i, sl] * t3_v[i, sl]
                acc = acc + h2_v[i, sl] * rr2_v[i, sl] * t2_v[i, sl]
                acc = acc + h3_v[i, sl] * rr3_v[i, sl] * t1_v[i, sl]
            s = jnp.sum(acc)
            plsc.store_scatter(out_v,
                               [jnp.full((L,), off, jnp.int32) + i],
                               jnp.full((L,), s, jnp.float32),
                               mask=lane0)
            return 0

        lax.fori_loop(0, C, row_body, 0)

    pltpu.sync_copy(out_v, out_hbm.at[pl.ds(base, RPW)])


@jax.jit
def _tri_vec(h_idx, r_idx, t_idx, ent_1, ent_2, ent_3, rel_1, rel_2, rel_3):
    mesh = plsc.VectorSubcoreMesh(core_axis_name="c", subcore_axis_name="s",
                                  num_cores=NC, num_subcores=NS)
    f = pl.kernel(
        _tri_vec_body,
        out_type=jax.ShapeDtypeStruct((BATCH,), jnp.float32),
        mesh=mesh,
        scratch_types=[
            pltpu.VMEM((RPW,), jnp.int32),
            pltpu.VMEM((RPW,), jnp.int32),
            pltpu.VMEM((RPW,), jnp.int32),
        ] + [pltpu.VMEM((C, PDIM), jnp.float32)] * 9 + [
            pltpu.VMEM((RPW,), jnp.float32),
            pltpu.SemaphoreType.DMA,
        ],
        compiler_params=pltpu.CompilerParams(needs_layout_passes=False),
    )
    return f(h_idx, r_idx, t_idx,
             _pad_rows(ent_1), _pad_rows(ent_2), _pad_rows(ent_3),
             _pad_rows(rel_1), _pad_rows(rel_2), _pad_rows(rel_3))


def kernel(data, ent_1, ent_2, ent_3, rel_1, rel_2, rel_3):
    h_idx = data[:, 0]
    r_idx = data[:, 1]
    t_idx = data[:, 2]
    return _tri_vec(h_idx, r_idx, t_idx, ent_1, ent_2, ent_3,
                    rel_1, rel_2, rel_3)


# speedup vs baseline: 2.9151x; 1.0659x over previous
import functools

import jax
import jax.numpy as jnp
from jax import lax
from jax.experimental import pallas as pl
from jax.experimental.pallas import tpu as pltpu
from jax.experimental.pallas import tpu_sc as plsc

NC = 2
NS = 16
NW = NC * NS
L = 16

BATCH = 16384
DIM = 64
PDIM = 128
NROW = 100000
RPW = BATCH // NW
C = 128
NCHUNK = RPW // C

TBLK = 4096


def _transpose_body(in_ref, out_ref):
    rows = lax.broadcasted_iota(jnp.int32, (DIM, PDIM), 0)
    cols = lax.broadcasted_iota(jnp.int32, (DIM, PDIM), 1)
    eye = (rows == cols).astype(jnp.float32)
    out_ref[...] = lax.dot_general(
        in_ref[...], eye, (((0,), (0,)), ((), ())),
        preferred_element_type=jnp.float32)


def _pad_rows(table):
    nblk = (NROW + TBLK - 1) // TBLK
    return pl.pallas_call(
        _transpose_body,
        grid=(nblk,),
        in_specs=[pl.BlockSpec((DIM, TBLK), lambda i: (0, i))],
        out_specs=pl.BlockSpec((TBLK, PDIM), lambda i: (i, 0)),
        out_shape=jax.ShapeDtypeStruct((NROW, PDIM), jnp.float32),
    )(table.T)


def _term_body(hidx_hbm, ridx_hbm, tidx_hbm,
               a_hbm, b_hbm, c_hbm,
               out_hbm,
               hid_v, rid_v, tid_v,
               a_v, b_v, c_v,
               out_v, sem):
    wid = lax.axis_index("s") * NC + lax.axis_index("c")
    base = wid * RPW

    pltpu.sync_copy(hidx_hbm.at[pl.ds(base, RPW)], hid_v)
    pltpu.sync_copy(ridx_hbm.at[pl.ds(base, RPW)], rid_v)
    pltpu.sync_copy(tidx_hbm.at[pl.ds(base, RPW)], tid_v)

    lanes = lax.iota(jnp.int32, L)
    lane0 = lanes == 0

    for c in range(NCHUNK):
        off = c * C
        copies = [
            pltpu.async_copy(a_hbm.at[hid_v.at[pl.ds(off, C)]], a_v, sem),
            pltpu.async_copy(b_hbm.at[rid_v.at[pl.ds(off, C)]], b_v, sem),
            pltpu.async_copy(c_hbm.at[tid_v.at[pl.ds(off, C)]], c_v, sem),
        ]
        for cp in copies:
            cp.wait()

        def row_body(i, _, off=off):
            acc = jnp.zeros((L,), jnp.float32)
            for k in range(DIM // L):
                sl = pl.ds(k * L, L)
                acc = acc + a_v[i, sl] * b_v[i, sl] * c_v[i, sl]
            s = jnp.sum(acc)
            plsc.store_scatter(out_v,
                               [jnp.full((L,), off, jnp.int32) + i],
                               jnp.full((L,), s, jnp.float32),
                               mask=lane0)
            return 0

        lax.fori_loop(0, C, row_body, 0)

    pltpu.sync_copy(out_v, out_hbm.at[pl.ds(base, RPW)])


def _term(h_idx, r_idx, t_idx, a, b, c):
    mesh = plsc.VectorSubcoreMesh(core_axis_name="c", subcore_axis_name="s",
                                  num_cores=NC, num_subcores=NS)
    f = pl.kernel(
        _term_body,
        out_type=jax.ShapeDtypeStruct((BATCH,), jnp.float32),
        mesh=mesh,
        scratch_types=[
            pltpu.VMEM((RPW,), jnp.int32),
            pltpu.VMEM((RPW,), jnp.int32),
            pltpu.VMEM((RPW,), jnp.int32),
        ] + [pltpu.VMEM((C, PDIM), jnp.float32)] * 3 + [
            pltpu.VMEM((RPW,), jnp.float32),
            pltpu.SemaphoreType.DMA,
        ],
        compiler_params=pltpu.CompilerParams(needs_layout_passes=False),
    )
    return f(h_idx, r_idx, t_idx, a, b, c)


@jax.jit
def _tri_vec(h_idx, r_idx, t_idx, ent_1, ent_2, ent_3, rel_1, rel_2, rel_3):
    pe1 = _pad_rows(ent_1)
    pr1 = _pad_rows(rel_1)
    pe3 = _pad_rows(ent_3)
    s1 = _term(h_idx, r_idx, t_idx, pe1, pr1, pe3)
    pe2 = _pad_rows(ent_2)
    pr2 = _pad_rows(rel_2)
    s2 = _term(h_idx, r_idx, t_idx, pe2, pr2, pe2)
    pr3 = _pad_rows(rel_3)
    s3 = _term(h_idx, r_idx, t_idx, pe3, pr3, pe1)
    return s1 + s2 + s3


def kernel(data, ent_1, ent_2, ent_3, rel_1, rel_2, rel_3):
    h_idx = data[:, 0]
    r_idx = data[:, 1]
    t_idx = data[:, 2]
    return _tri_vec(h_idx, r_idx, t_idx, ent_1, ent_2, ent_3,
                    rel_1, rel_2, rel_3)
